# baseline (speedup 1.0000x reference)
import functools

import jax
import jax.numpy as jnp
from jax import lax
from jax.experimental import pallas as pl
from jax.experimental.pallas import tpu as pltpu
from jax.experimental.pallas import tpu_sc as plsc

NUM_NODES = 1000000
D = 64
B = 16384
NC = 2
NS = 16
L = 16
NW = NC * NS
BPW = B // NW
ST = L
NSTG = BPW // ST
G = 8


def _body(I_hbm, J_hbm, node_hbm, ctx_hbm, out_hbm,
          idx_i, idx_j, bufs_i, bufs_j, out_v,
          sem_a0, sem_a1, sem_a2, sem_a3,
          sem_b0, sem_b1, sem_b2, sem_b3):
    wid = lax.axis_index("s") * NC + lax.axis_index("c")
    base = wid * BPW

    pltpu.sync_copy(I_hbm.at[wid], idx_i)
    pltpu.sync_copy(J_hbm.at[wid], idx_j)

    sems = [[sem_a0, sem_a1, sem_a2, sem_a3],
            [sem_b0, sem_b1, sem_b2, sem_b3]]
    lanes = lax.iota(jnp.int32, L)
    seven = jnp.full((L,), 7, jnp.int32)

    def fire(s, par):
        gi = lax.shift_right_logical(idx_i[pl.ds(s * L, L)], 3)
        gj = lax.shift_right_logical(idx_j[pl.ds(s * L, L)], 3)
        for k in range(L):
            pltpu.async_copy(node_hbm.at[gi[k]], bufs_i.at[par, k],
                             sems[par][k % 4])
            pltpu.async_copy(ctx_hbm.at[gj[k]], bufs_j.at[par, k],
                             sems[par][(k + 1) % 4])

    def drain(par):
        for q in range(4):
            pltpu.make_async_copy(node_hbm.at[pl.ds(0, ST // 4)],
                                  bufs_i.at[par, pl.ds(0, ST // 4)],
                                  sems[par][q]).wait()
            pltpu.make_async_copy(node_hbm.at[pl.ds(0, ST // 4)],
                                  bufs_j.at[par, pl.ds(0, ST // 4)],
                                  sems[par][q]).wait()

    def compute(s, par):
        sub_i = idx_i[pl.ds(s * L, L)] & seven
        sub_j = idx_j[pl.ds(s * L, L)] & seven
        buf_i = bufs_i.at[par]
        buf_j = bufs_j.at[par]

        def dstep(d, acc):
            col = jnp.zeros((L,), jnp.int32) + d
            vi = plsc.load_gather(buf_i, [lanes, sub_i, col])
            vj = plsc.load_gather(buf_j, [lanes, sub_j, col])
            return acc + vi * vj

        acc = lax.fori_loop(0, D, dstep, jnp.zeros((L,), jnp.float32))
        out_v[pl.ds(s * L, L)] = acc

    fire(0, 0)
    for s in range(NSTG):
        par = s % 2
        if s + 1 < NSTG:
            fire(s + 1, 1 - par)
        drain(par)
        compute(s, par)

    pltpu.sync_copy(out_v, out_hbm.at[pl.ds(base, BPW)])


@jax.jit
def _line_second(I2, J2, node3, ctx3):
    kern = functools.partial(
        pl.kernel,
        out_type=jax.ShapeDtypeStruct((B,), jnp.float32),
        mesh=plsc.VectorSubcoreMesh(core_axis_name="c", subcore_axis_name="s"),
        compiler_params=pltpu.CompilerParams(needs_layout_passes=False),
        scratch_types=[
            pltpu.VMEM((BPW,), jnp.int32),
            pltpu.VMEM((BPW,), jnp.int32),
            pltpu.VMEM((2, ST, G, D), jnp.float32),
            pltpu.VMEM((2, ST, G, D), jnp.float32),
            pltpu.VMEM((BPW,), jnp.float32),
            pltpu.SemaphoreType.DMA,
            pltpu.SemaphoreType.DMA,
            pltpu.SemaphoreType.DMA,
            pltpu.SemaphoreType.DMA,
            pltpu.SemaphoreType.DMA,
            pltpu.SemaphoreType.DMA,
            pltpu.SemaphoreType.DMA,
            pltpu.SemaphoreType.DMA,
        ],
    )(_body)
    return kern(I2, J2, node3, ctx3)


def kernel(I, J, node_emb, context_emb):
    I2 = I.astype(jnp.int32).reshape(NW, BPW)
    J2 = J.astype(jnp.int32).reshape(NW, BPW)
    node3 = node_emb.reshape(NUM_NODES // G, G, D)
    ctx3 = context_emb.reshape(NUM_NODES // G, G, D)
    return _line_second(I2, J2, node3, ctx3)
